# baseline (speedup 1.0000x reference)
import functools

import jax
import jax.numpy as jnp
from jax import lax
from jax.experimental import pallas as pl
from jax.experimental.pallas import tpu as pltpu
from jax.experimental.pallas import tpu_sc as plsc

NC = 2
NS = 16
NW = NC * NS
LANES = 128
DEGW = 16


def _pad_up(x, m):
    return (x + m - 1) // m * m


def _make_deg(npad, nrows, d):
    rpt = nrows // NW
    per_tile = npad // NS
    mesh = plsc.VectorSubcoreMesh(core_axis_name="c", subcore_axis_name="s")

    @functools.partial(
        pl.kernel,
        mesh=mesh,
        out_type=jax.ShapeDtypeStruct((NC, npad, d), jnp.float32),
        scratch_types=[
            pltpu.VMEM((rpt, LANES), jnp.int32),
            pltpu.VMEM((LANES, d), jnp.float32),
            pltpu.VMEM_SHARED((npad, d), jnp.float32),
            [pltpu.SemaphoreType.DMA] * 4,
        ],
    )
    def deg_kernel(ones2d, gidx, out, idx_a, ones_v, deg_sh, sems):
        c = lax.axis_index("c")
        s = lax.axis_index("s")
        wid = s * NC + c
        base = s * per_tile
        pltpu.sync_copy(ones2d.at[pl.ds(base, per_tile)],
                        deg_sh.at[pl.ds(base, per_tile)])
        pltpu.sync_copy(ones2d.at[pl.ds(0, LANES)], ones_v)
        pltpu.sync_copy(gidx.at[pl.ds(wid * rpt, rpt)], idx_a)
        plsc.subcore_barrier()

        def step(js, carry):
            j0 = 4 * js
            plan = [idx_a.at[j0 + k] for k in range(4)]
            ss = [pltpu.async_copy(ones_v, deg_sh.at[si], sems[k], add=True)
                  for k, si in enumerate(plan)]
            for sd in ss:
                sd.wait()
            return carry

        lax.fori_loop(0, rpt // 4, step, 0)
        plsc.subcore_barrier()
        pltpu.sync_copy(deg_sh.at[pl.ds(base, per_tile)],
                        out.at[c, pl.ds(base, per_tile)])

    return deg_kernel


def _make_agg(npad, nrows, d, r0, r1):
    assert 16 * (r0 + r1) == nrows and r0 % 16 == 0 and r1 % 16 == 0
    hrmax = max(r0, r1) // 2
    per_tile = npad // NS
    mesh = plsc.VectorSubcoreMesh(core_axis_name="c", subcore_axis_name="s")

    @functools.partial(
        pl.kernel,
        mesh=mesh,
        out_type=jax.ShapeDtypeStruct((NC, npad, d), jnp.float32),
        scratch_types=[
            pltpu.VMEM((hrmax, LANES), jnp.int32),
            pltpu.VMEM((hrmax, LANES), jnp.int32),
            pltpu.VMEM((2, LANES, d), jnp.float32),
            pltpu.VMEM_SHARED((npad, d), jnp.float32),
            [pltpu.SemaphoreType.DMA] * 6,
        ],
    )
    def agg_kernel(g, gidx, sidx, out, idx_ga, idx_sa, bufs, acc_sh, sems):
        c = lax.axis_index("c")
        s = lax.axis_index("s")
        base = s * per_tile
        rv = jnp.where(c == 0, r0, r1)
        row0 = c * (NS * r0) + s * rv
        hrpt = rv // 2
        nst = hrpt // 2
        pltpu.sync_copy(g.at[pl.ds(base, per_tile)],
                        acc_sh.at[pl.ds(base, per_tile)])
        plsc.subcore_barrier()

        HL = LANES // 2

        def issue_gather(j, k):
            for h in range(2):
                pltpu.async_copy(g.at[idx_ga.at[j, pl.ds(h * HL, HL)]],
                                 bufs.at[k, pl.ds(h * HL, HL)],
                                 sems[2 * k + h])

        def wait_gather(j, k):
            for h in range(2):
                pltpu.make_async_copy(g.at[idx_ga.at[j, pl.ds(h * HL, HL)]],
                                      bufs.at[k, pl.ds(h * HL, HL)],
                                      sems[2 * k + h]).wait()

        def step(js, carry):
            j0 = 2 * js
            ss = []
            for k in range(2):
                j = j0 + k
                wait_gather(j, k)
                ss.append(pltpu.async_copy(bufs.at[k], acc_sh.at[idx_sa.at[j]],
                                           sems[4 + k], add=True))
            for k in range(2):
                ss[k].wait()

                @pl.when(js < nst - 1)
                def _():
                    issue_gather(j0 + 2 + k, k)
            return carry

        for half in range(2):
            start = pl.multiple_of(row0 + half * hrpt, 8)
            pltpu.sync_copy(gidx.at[pl.ds(start, hrmax)], idx_ga)
            pltpu.sync_copy(sidx.at[pl.ds(start, hrmax)], idx_sa)
            for k in range(2):
                issue_gather(k, k)
            lax.fori_loop(0, nst, step, 0)
        plsc.subcore_barrier()
        pltpu.sync_copy(acc_sh.at[pl.ds(base, per_tile)],
                        out.at[c, pl.ds(base, per_tile)])

    return agg_kernel


def _dinv_of(dg_ref):
    deg = dg_ref[:, 0:1] + dg_ref[:, 1:2] - 1.0
    return lax.rsqrt(deg)


def _g_body(x_ref, w_ref, dg_ref, g_ref):
    dinv = _dinv_of(dg_ref)
    h = jnp.dot(x_ref[...], w_ref[...], preferred_element_type=jnp.float32)
    g_ref[...] = h * dinv


def _fin1_body(p_ref, g_ref, dg_ref, b_ref, w_ref, x1_ref, g2_ref):
    dinv = _dinv_of(dg_ref)
    t = (p_ref[0] + p_ref[1] - g_ref[...]) * dinv + b_ref[...]
    x1 = jnp.maximum(t, 0.01 * t)
    x1_ref[...] = x1
    g2_ref[...] = jnp.dot(x1, w_ref[...],
                          preferred_element_type=jnp.float32) * dinv


def _fin2_body(p_ref, g_ref, dg_ref, b_ref, x2_ref):
    dinv = _dinv_of(dg_ref)
    t = (p_ref[0] + p_ref[1] - g_ref[...]) * dinv + b_ref[...]
    x2_ref[...] = jnp.maximum(t, 0.01 * t)


def kernel(id_embedding, edge_index, W1, b1, W2, b2):
    n, d = id_embedding.shape
    e = edge_index.shape[1]
    npad = _pad_up(n + 1, NS * 16)
    epad = _pad_up(e, LANES * NW)
    nrows = epad // LANES
    blk = 1280
    grid = npad // blk

    ei = edge_index.astype(jnp.int32)
    pad_e = jnp.full((epad - e,), n, jnp.int32)
    r0, r1 = 112, 48
    hrmax = max(r0, r1) // 2
    pad_r = jnp.full((hrmax * LANES,), n, jnp.int32)
    gidx = jnp.concatenate([ei[0], pad_e, ei[1], pad_e, pad_r]).reshape(
        2 * nrows + hrmax, LANES)
    sidx = jnp.concatenate([ei[1], pad_e, ei[0], pad_e, pad_r]).reshape(
        2 * nrows + hrmax, LANES)
    xpad = jnp.zeros((npad, d), jnp.float32).at[:n].set(id_embedding)

    ones2d = jnp.ones((npad, d), jnp.float32)
    degp = _make_deg(npad, 2 * nrows, d)(ones2d, gidx)
    degt = jnp.concatenate([degp[0, :, :1], degp[1, :, :1]], axis=1)

    row_spec = pl.BlockSpec((blk, d), lambda i: (i, 0))
    w_spec = pl.BlockSpec((d, d), lambda i: (0, 0))
    dg_spec = pl.BlockSpec((blk, 2), lambda i: (i, 0))
    b_spec = pl.BlockSpec((1, d), lambda i: (0, 0))
    p_spec = pl.BlockSpec((NC, blk, d), lambda i: (0, i, 0))
    row_shape = jax.ShapeDtypeStruct((npad, d), jnp.float32)

    g1 = pl.pallas_call(
        _g_body, grid=(grid,),
        in_specs=[row_spec, w_spec, dg_spec],
        out_specs=row_spec, out_shape=row_shape,
    )(xpad, W1, degt)

    agg = _make_agg(npad, 2 * nrows, d, r0, r1)
    p1 = agg(g1, gidx, sidx)

    x1, g2 = pl.pallas_call(
        _fin1_body, grid=(grid,),
        in_specs=[p_spec, row_spec, dg_spec, b_spec, w_spec],
        out_specs=[row_spec, row_spec], out_shape=[row_shape, row_shape],
    )(p1, g1, degt, b1.reshape(1, d), W2)

    p2 = agg(g2, gidx, sidx)

    x2 = pl.pallas_call(
        _fin2_body, grid=(grid,),
        in_specs=[p_spec, row_spec, dg_spec, b_spec],
        out_specs=row_spec, out_shape=row_shape,
    )(p2, g2, degt, b2.reshape(1, d))

    return jnp.concatenate([x1[:n], x2[:n]], axis=1)


# speedup vs baseline: 1.1316x; 1.1316x over previous
import functools

import jax
import jax.numpy as jnp
from jax import lax
from jax.experimental import pallas as pl
from jax.experimental.pallas import tpu as pltpu
from jax.experimental.pallas import tpu_sc as plsc

NC = 2
NS = 16
NW = NC * NS
LANES = 128
DEGW = 16


def _pad_up(x, m):
    return (x + m - 1) // m * m


def _make_deg(npad, nrows, d):
    rpt = nrows // NW
    per_tile = npad // NS
    mesh = plsc.VectorSubcoreMesh(core_axis_name="c", subcore_axis_name="s")

    @functools.partial(
        pl.kernel,
        mesh=mesh,
        out_type=jax.ShapeDtypeStruct((NC, npad, d), jnp.float32),
        scratch_types=[
            pltpu.VMEM((rpt, LANES), jnp.int32),
            pltpu.VMEM((LANES, d), jnp.float32),
            pltpu.VMEM_SHARED((npad, d), jnp.float32),
            [pltpu.SemaphoreType.DMA] * 4,
        ],
    )
    def deg_kernel(ones2d, gidx, out, idx_a, ones_v, deg_sh, sems):
        c = lax.axis_index("c")
        s = lax.axis_index("s")
        wid = s * NC + c
        base = s * per_tile
        pltpu.sync_copy(ones2d.at[pl.ds(base, per_tile)],
                        deg_sh.at[pl.ds(base, per_tile)])
        pltpu.sync_copy(ones2d.at[pl.ds(0, LANES)], ones_v)
        pltpu.sync_copy(gidx.at[pl.ds(wid * rpt, rpt)], idx_a)
        plsc.subcore_barrier()

        def step(js, carry):
            j0 = 4 * js
            plan = [idx_a.at[j0 + k] for k in range(4)]
            ss = [pltpu.async_copy(ones_v, deg_sh.at[si], sems[k], add=True)
                  for k, si in enumerate(plan)]
            for sd in ss:
                sd.wait()
            return carry

        lax.fori_loop(0, rpt // 4, step, 0)
        plsc.subcore_barrier()
        pltpu.sync_copy(deg_sh.at[pl.ds(base, per_tile)],
                        out.at[c, pl.ds(base, per_tile)])

    return deg_kernel


def _make_agg(npad, nrows, d, r0, r1):
    assert 16 * (r0 + r1) == nrows and r0 % 16 == 0 and r1 % 16 == 0
    hrmax = max(r0, r1) // 2
    per_tile = npad // NS
    mesh = plsc.VectorSubcoreMesh(core_axis_name="c", subcore_axis_name="s")

    @functools.partial(
        pl.kernel,
        mesh=mesh,
        out_type=jax.ShapeDtypeStruct((NC, npad, d), jnp.float32),
        scratch_types=[
            pltpu.VMEM((hrmax, LANES), jnp.int32),
            pltpu.VMEM((hrmax, LANES), jnp.int32),
            pltpu.VMEM((2, LANES, d), jnp.float32),
            pltpu.VMEM_SHARED((npad, d), jnp.float32),
            [pltpu.SemaphoreType.DMA] * 6,
        ],
    )
    def agg_kernel(g, gidx, sidx, out, idx_ga, idx_sa, bufs, acc_sh, sems):
        c = lax.axis_index("c")
        s = lax.axis_index("s")
        base = s * per_tile
        rv = jnp.where(c == 0, r0, r1)
        row0 = c * (NS * r0) + s * rv
        hrpt = rv // 2
        nst = hrpt // 2
        pltpu.sync_copy(g.at[pl.ds(base, per_tile)],
                        acc_sh.at[pl.ds(base, per_tile)])
        plsc.subcore_barrier()

        HL = LANES // 2

        def issue_gather(j, k):
            for h in range(2):
                pltpu.async_copy(g.at[idx_ga.at[j, pl.ds(h * HL, HL)]],
                                 bufs.at[k, pl.ds(h * HL, HL)],
                                 sems[2 * k + h])

        def wait_gather(j, k):
            for h in range(2):
                pltpu.make_async_copy(g.at[idx_ga.at[j, pl.ds(h * HL, HL)]],
                                      bufs.at[k, pl.ds(h * HL, HL)],
                                      sems[2 * k + h]).wait()

        def step(js, carry):
            j0 = 2 * js
            ss = []
            for k in range(2):
                j = j0 + k
                wait_gather(j, k)
                ss.append(pltpu.async_copy(bufs.at[k], acc_sh.at[idx_sa.at[j]],
                                           sems[4 + k], add=True))
            for k in range(2):
                ss[k].wait()

                @pl.when(js < nst - 1)
                def _():
                    issue_gather(j0 + 2 + k, k)
            return carry

        for half in range(2):
            start = pl.multiple_of(row0 + half * hrpt, 8)
            pltpu.sync_copy(gidx.at[pl.ds(start, hrmax)], idx_ga)
            pltpu.sync_copy(sidx.at[pl.ds(start, hrmax)], idx_sa)
            for k in range(2):
                issue_gather(k, k)
            lax.fori_loop(0, nst, step, 0)
        plsc.subcore_barrier()
        pltpu.sync_copy(acc_sh.at[pl.ds(base, per_tile)],
                        out.at[c, pl.ds(base, per_tile)])

    return agg_kernel


def _dinv_of(dg_ref):
    deg = dg_ref[:, 0:1] + dg_ref[:, 1:2] - 1.0
    return lax.rsqrt(deg)


def _g_body(x_ref, w_ref, dg_ref, g_ref):
    dinv = _dinv_of(dg_ref)
    h = jnp.dot(x_ref[...], w_ref[...], preferred_element_type=jnp.float32)
    g_ref[...] = h * dinv


def _fin1_body(p_ref, g_ref, dg_ref, b_ref, w_ref, x1_ref, g2_ref):
    dinv = _dinv_of(dg_ref)
    t = (p_ref[0] + p_ref[1] - g_ref[...]) * dinv + b_ref[...]
    x1 = jnp.maximum(t, 0.01 * t)
    x1_ref[...] = x1
    g2_ref[...] = jnp.dot(x1, w_ref[...],
                          preferred_element_type=jnp.float32) * dinv


def _fin2_body(p_ref, g_ref, dg_ref, b_ref, x2_ref):
    dinv = _dinv_of(dg_ref)
    t = (p_ref[0] + p_ref[1] - g_ref[...]) * dinv + b_ref[...]
    x2_ref[...] = jnp.maximum(t, 0.01 * t)


def kernel(id_embedding, edge_index, W1, b1, W2, b2):
    n, d = id_embedding.shape
    e = edge_index.shape[1]
    npad = _pad_up(n + 1, NS * 16)
    epad = _pad_up(e, LANES * NW)
    nrows = epad // LANES
    blk = 1280
    grid = npad // blk

    ei = edge_index.astype(jnp.int32)
    pad_e = jnp.full((epad - e,), n, jnp.int32)
    r0, r1 = 80, 80
    hrmax = max(r0, r1) // 2
    pad_r = jnp.full((hrmax * LANES,), n, jnp.int32)
    gidx = jnp.concatenate([ei[0], pad_e, ei[1], pad_e, pad_r]).reshape(
        2 * nrows + hrmax, LANES)
    sidx = jnp.concatenate([ei[1], pad_e, ei[0], pad_e, pad_r]).reshape(
        2 * nrows + hrmax, LANES)
    xpad = jnp.zeros((npad, d), jnp.float32).at[:n].set(id_embedding)

    ones2d = jnp.ones((npad, d), jnp.float32)
    degp = _make_deg(npad, 2 * nrows, d)(ones2d, gidx)
    degt = jnp.concatenate([degp[0, :, :1], degp[1, :, :1]], axis=1)

    row_spec = pl.BlockSpec((blk, d), lambda i: (i, 0))
    w_spec = pl.BlockSpec((d, d), lambda i: (0, 0))
    dg_spec = pl.BlockSpec((blk, 2), lambda i: (i, 0))
    b_spec = pl.BlockSpec((1, d), lambda i: (0, 0))
    p_spec = pl.BlockSpec((NC, blk, d), lambda i: (0, i, 0))
    row_shape = jax.ShapeDtypeStruct((npad, d), jnp.float32)

    g1 = pl.pallas_call(
        _g_body, grid=(grid,),
        in_specs=[row_spec, w_spec, dg_spec],
        out_specs=row_spec, out_shape=row_shape,
    )(xpad, W1, degt)

    agg = _make_agg(npad, 2 * nrows, d, r0, r1)
    p1 = agg(g1, gidx, sidx)

    x1, g2 = pl.pallas_call(
        _fin1_body, grid=(grid,),
        in_specs=[p_spec, row_spec, dg_spec, b_spec, w_spec],
        out_specs=[row_spec, row_spec], out_shape=[row_shape, row_shape],
    )(p1, g1, degt, b1.reshape(1, d), W2)

    p2 = agg(g2, gidx, sidx)

    x2 = pl.pallas_call(
        _fin2_body, grid=(grid,),
        in_specs=[p_spec, row_spec, dg_spec, b_spec],
        out_specs=row_spec, out_shape=row_shape,
    )(p2, g2, degt, b2.reshape(1, d))

    return jnp.concatenate([x1[:n], x2[:n]], axis=1)
